# baseline (speedup 1.0000x reference)
import math
import functools

import jax
import jax.numpy as jnp
from jax import lax
from jax.experimental import pallas as pl
from jax.experimental.pallas import tpu as pltpu
from jax.experimental.pallas import tpu_sc as plsc

_VOCAB = 256
_D = 2048
_SCALE = math.sqrt(_D)

_NC = 2
_NS = 16
_NW = _NC * _NS

_C = 16


def _prescale_body(t_ref, o_ref):
    row = lax.broadcasted_iota(jnp.int32, (_VOCAB, _D), 0)
    o_ref[...] = jnp.where(row == 0, 0.0, t_ref[...] * _SCALE)


def _prescale(table):
    return pl.pallas_call(
        _prescale_body,
        out_shape=jax.ShapeDtypeStruct((_VOCAB, _D), jnp.float32),
    )(table)


def _gather_body(tbl_hbm, idx_hbm, out_hbm, idx_v, buf0, buf1, buf2,
                 gs0, gs1, gs2, ss0, ss1, ss2):
    wid = lax.axis_index("s") * _NC + lax.axis_index("c")
    bpw = idx_hbm.shape[0] // _NW
    base = wid * bpw
    nch = bpw // _C
    bufs = (buf0, buf1, buf2)
    gsems = (gs0, gs1, gs2)
    ssems = (ss0, ss1, ss2)

    pltpu.sync_copy(idx_hbm.at[pl.ds(base, bpw)], idx_v)

    def gather_start(c, b):
        iv = idx_v[pl.ds(c * _C, _C)]
        pltpu.async_copy(tbl_hbm.at[iv], bufs[b], gsems[b])

    def gather_wait(b):
        iv = idx_v[pl.ds(0, _C)]
        pltpu.make_async_copy(tbl_hbm.at[iv], bufs[b], gsems[b]).wait()

    def scatter_start(c, b):
        pltpu.async_copy(bufs[b], out_hbm.at[pl.ds(base + c * _C, _C)],
                         ssems[b])

    def scatter_wait(b):
        pltpu.make_async_copy(bufs[b], out_hbm.at[pl.ds(base, _C)],
                              ssems[b]).wait()

    def step(c, j, skip_free):
        nb2 = (j + 2) % 3
        if skip_free:
            gather_start(c + 2, nb2)
        else:
            @pl.when(c + 2 < nch)
            def _():
                scatter_wait(nb2)
                gather_start(c + 2, nb2)

        gather_wait(j)
        scatter_start(c, j)

    gather_start(0, 0)
    gather_start(1, 1)

    step(0, 0, True)
    step(1, 1, False)
    step(2, 2, False)

    def group(g, carry):
        for j in range(3):
            step(g + j, j, False)
        return carry

    lax.fori_loop(1, nch // 3, lambda i, cr: group(i * 3, cr), 0)
    for c in range(3 * (nch // 3), nch):
        step(c, c % 3, False)
    for j in range(3):
        scatter_wait(j)


def _gather(table_eff, idx):
    n = idx.shape[0]
    bpw = n // _NW
    mesh = plsc.VectorSubcoreMesh(core_axis_name="c", subcore_axis_name="s")
    return pl.kernel(
        _gather_body,
        out_type=jax.ShapeDtypeStruct((n, _D), jnp.float32),
        mesh=mesh,
        scratch_types=[
            pltpu.VMEM((bpw,), jnp.int32),
            pltpu.VMEM((_C, _D), jnp.float32),
            pltpu.VMEM((_C, _D), jnp.float32),
            pltpu.VMEM((_C, _D), jnp.float32),
            pltpu.SemaphoreType.DMA,
            pltpu.SemaphoreType.DMA,
            pltpu.SemaphoreType.DMA,
            pltpu.SemaphoreType.DMA,
            pltpu.SemaphoreType.DMA,
            pltpu.SemaphoreType.DMA,
        ],
    )(table_eff, idx)


@jax.jit
def kernel(x, table):
    b, s = x.shape
    idx = x.reshape(-1).astype(jnp.int32)
    table_eff = _prescale(table)
    out = _gather(table_eff, idx)
    return out.reshape(b, s, _D)
